# Initial kernel scaffold; baseline (speedup 1.0000x reference)
import jax
import jax.numpy as jnp
from jax.experimental import pallas as pl


def kernel(fourmomenta, W_msg, b_msg, W_upd, b_upd, W_out, b_out):
    raise NotImplementedError("write your pallas kernel here")


# speedup vs baseline: 428.4131x; 428.4131x over previous
import functools

import jax
import jax.numpy as jnp
import numpy as np
from jax.experimental import pallas as pl

B = 16384
N = 10
HIDDEN = 64
PT = np.array([0, 0, 1, 1, 2, 2, 2, 3, 3, 3], dtype=np.int32)
METRIC = np.array([1.0, -1.0, -1.0, -1.0], dtype=np.float32)

E_BLK = 512


def _graph_block(p2_ref, wsrc_ref, wdst_ref, wupd_ref, ws_ref, sred_ref,
                 cmsg_ref, u0_ref, cupd_ref, wm_ref, wo_ref, bo_ref, out_ref):
    p2 = p2_ref[...]
    lane = jax.lax.broadcasted_iota(jnp.int32, (1, N * 4), 1)
    metric40 = jnp.where(lane % 4 == 0, 1.0, -1.0)
    q2 = p2 * metric40
    pq = p2 * q2
    d10 = jnp.dot(pq, sred_ref[...], preferred_element_type=jnp.float32)

    u_all = jnp.dot(p2, wsrc_ref[...], preferred_element_type=jnp.float32) + u0_ref[...]
    v_all = jnp.dot(p2, wdst_ref[...], preferred_element_type=jnp.float32) + cmsg_ref[...]

    acc = jnp.zeros_like(v_all)
    diags = []
    for i in range(N):
        q_i = q2[:, 4 * i:4 * i + 4]
        g_i = jnp.dot(p2 * jnp.concatenate([q_i] * N, axis=1), sred_ref[...],
                      preferred_element_type=jnp.float32)
        s_i = d10 + d10[:, i:i + 1] + 2.0 * g_i
        sflat = jnp.dot(s_i, ws_ref[...], preferred_element_type=jnp.float32)
        u_i = u_all[:, HIDDEN * i:HIDDEN * (i + 1)]
        m_i = jnp.maximum(jnp.concatenate([u_i] * N, axis=1) + v_all + sflat, 0.0)
        acc = acc + m_i
        diags.append(m_i[:, HIDDEN * i:HIDDEN * (i + 1)])
    agg = (acc - jnp.concatenate(diags, axis=1)) * (1.0 / 9.0)

    pupd = jnp.dot(p2, wupd_ref[...], preferred_element_type=jnp.float32) + cupd_ref[...]
    wm = wm_ref[...]
    hacc = jnp.zeros((p2.shape[0], HIDDEN), dtype=jnp.float32)
    for j in range(N):
        h_j = jnp.maximum(
            jnp.dot(agg[:, HIDDEN * j:HIDDEN * (j + 1)], wm,
                    preferred_element_type=jnp.float32)
            + pupd[:, HIDDEN * j:HIDDEN * (j + 1)], 0.0)
        hacc = hacc + h_j
    amp = jnp.sum(hacc * wo_ref[...], axis=1, keepdims=True) * (1.0 / N) + bo_ref[...]
    out_ref[...] = amp


@functools.partial(jax.jit, static_argnames=())
def _amp(fourmomenta, W_msg, b_msg, W_upd, b_upd, W_out, b_out):
    f32 = jnp.float32
    p2 = fourmomenta.reshape(B, N * 4).astype(f32)

    i10 = jnp.eye(N, dtype=f32)
    onehot = jnp.eye(4, dtype=f32)[PT]
    wp_src, wp_dst = W_msg[4:8], W_msg[12:16]
    u0 = onehot @ W_msg[0:4]
    v0 = onehot @ W_msg[8:12]
    w_s = W_msg[16]
    wp_upd = W_upd[4:8]
    ha0 = onehot @ W_upd[0:4]
    w_m = W_upd[8:72]

    wsrc_blk = jnp.kron(i10, wp_src)
    wdst_blk = jnp.kron(i10, wp_dst)
    wupd_blk = jnp.kron(i10, wp_upd)
    ws_blk = jnp.kron(i10, w_s[None, :])
    sred = jnp.kron(i10, jnp.ones((4, 1), f32))
    u0f = u0.reshape(1, N * HIDDEN)
    cmsg = (v0 + b_msg[None, :]).reshape(1, N * HIDDEN)
    cupd = (ha0 + b_upd[None, :]).reshape(1, N * HIDDEN)
    wo = W_out.reshape(1, HIDDEN)
    bo = b_out.reshape(1, 1)

    grid = (B // E_BLK,)
    rep = lambda shape: pl.BlockSpec(shape, lambda i: (0, 0))
    amp = pl.pallas_call(
        _graph_block,
        grid=grid,
        in_specs=[
            pl.BlockSpec((E_BLK, N * 4), lambda i: (i, 0)),
            rep((N * 4, N * HIDDEN)),
            rep((N * 4, N * HIDDEN)),
            rep((N * 4, N * HIDDEN)),
            rep((N, N * HIDDEN)),
            rep((N * 4, N)),
            rep((1, N * HIDDEN)),
            rep((1, N * HIDDEN)),
            rep((1, N * HIDDEN)),
            rep((HIDDEN, HIDDEN)),
            rep((1, HIDDEN)),
            rep((1, 1)),
        ],
        out_specs=pl.BlockSpec((E_BLK, 1), lambda i: (i, 0)),
        out_shape=jax.ShapeDtypeStruct((B, 1), f32),
    )(p2, wsrc_blk, wdst_blk, wupd_blk, ws_blk, sred, cmsg, u0f, cupd,
      w_m, wo, bo)
    return amp


def kernel(fourmomenta, W_msg, b_msg, W_upd, b_upd, W_out, b_out):
    amp = _amp(fourmomenta, W_msg, b_msg, W_upd, b_upd, W_out, b_out)
    frames = jnp.broadcast_to(jnp.eye(4, dtype=fourmomenta.dtype)[None, None],
                              (B, N, 4, 4))
    tracker = jnp.zeros(())
    return (amp, tracker, frames)
